# baseline (speedup 1.0000x reference)
import functools

import jax
import jax.numpy as jnp
from jax import lax
from jax.experimental import pallas as pl
from jax.experimental.pallas import tpu as pltpu
from jax.experimental.pallas import tpu_sc as plsc

NUM_CHIPS = 8
EXPERTS_PER_CHIP = 8
N_EXPERTS = 64
TOP_K = 8
METADATA_LEN = 8
MAX_DISP = 256
SEQ = 128
HIDDEN = 2048

N_PICKS = NUM_CHIPS * SEQ * TOP_K
N_SLOTS = N_EXPERTS * MAX_DISP
LANES = 16
NW = 32
EPW = N_EXPERTS // NW
SLOTS_PW = EPW * MAX_DISP
X_ROWS = NUM_CHIPS * SEQ
SENTINEL = X_ROWS

_sc_mesh = plsc.VectorSubcoreMesh(core_axis_name="c", subcore_axis_name="s")


@functools.partial(
    pl.kernel,
    out_type=(
        jax.ShapeDtypeStruct((N_SLOTS,), jnp.int32),
        jax.ShapeDtypeStruct((N_SLOTS * METADATA_LEN,), jnp.int32),
        jax.ShapeDtypeStruct((NW, LANES), jnp.int32),
    ),
    mesh=_sc_mesh,
    compiler_params=pltpu.CompilerParams(needs_layout_passes=False),
    scratch_types=(
        pltpu.VMEM((N_PICKS,), jnp.int32),
        pltpu.VMEM((N_PICKS,), jnp.int32),
        pltpu.VMEM((SLOTS_PW,), jnp.int32),
        pltpu.VMEM((SLOTS_PW * METADATA_LEN,), jnp.int32),
        pltpu.VMEM((LANES,), jnp.int32),
        pltpu.VMEM((N_PICKS,), jnp.int32),
    ),
)
def _route_sc(idx_hbm, wb_hbm, inv_hbm, meta_hbm, cnt_hbm,
              idx_v, wb_v, inv_v, meta_v, cnt_v, pk_v):
    w = lax.axis_index("s") * 2 + lax.axis_index("c")
    pltpu.sync_copy(idx_hbm, idx_v)
    pltpu.sync_copy(wb_hbm, wb_v)

    sent = jnp.full((LANES,), SENTINEL, jnp.int32)
    neg1 = jnp.full((LANES,), -1, jnp.int32)
    zero = jnp.zeros((LANES,), jnp.int32)
    iota = lax.iota(jnp.int32, LANES)

    def init_inv(i, c):
        inv_v[pl.ds(i * jnp.int32(LANES), LANES)] = sent
        return c
    lax.fori_loop(jnp.int32(0), jnp.int32(SLOTS_PW // LANES), init_inv, 0)

    def init_meta(i, c):
        meta_v[pl.ds(i * jnp.int32(LANES), LANES)] = neg1
        return c
    lax.fori_loop(jnp.int32(0), jnp.int32(SLOTS_PW * METADATA_LEN // LANES),
                  init_meta, 0)

    e0 = lax.convert_element_type(w, jnp.int32) * jnp.int32(EPW)
    wid = lax.convert_element_type(w, jnp.int32)

    def scan_a(p, off):
        base0 = p * jnp.int32(4 * LANES)
        for u in range(4):
            base = base0 + jnp.int32(u * LANES)
            ev = idx_v[pl.ds(base, LANES)]
            mm = lax.shift_right_logical(ev, jnp.int32(1)) == wid
            mi = jnp.where(mm, jnp.int32(1), jnp.int32(0))
            pos = off + plsc.cumsum(mi) - mi
            plsc.store_scatter(pk_v, [pos], base + iota, mask=mm)
            off = off + plsc.all_reduce_population_count(mm)
        return off

    offv = lax.fori_loop(jnp.int32(0), jnp.int32(N_PICKS // LANES // 4),
                         scan_a, zero)
    n = lax.reduce_max(offv, axes=(0,))

    def scan_b(q, carry):
        r0, r1 = carry
        base = q * jnp.int32(LANES)
        vl = (base + iota) < n
        pv = pk_v[pl.ds(base, LANES)]
        ev = plsc.load_gather(idx_v, [pv], mask=vl)
        wv = plsc.load_gather(wb_v, [pv], mask=vl)
        m0r = vl & (ev == e0)
        m1r = vl & (ev == e0 + 1)
        i0 = jnp.where(m0r, jnp.int32(1), jnp.int32(0))
        i1 = jnp.where(m1r, jnp.int32(1), jnp.int32(0))
        s0 = r0 + plsc.cumsum(i0) - i0
        s1 = r1 + plsc.cumsum(i1) - i1
        m0 = m0r & (s0 < MAX_DISP)
        m1 = m1r & (s1 < MAX_DISP)
        mm = m0 | m1
        local = jnp.where(m0, s0, s1 + MAX_DISP)
        sh3 = jnp.int32(3)
        plsc.store_scatter(inv_v, [local], lax.shift_right_logical(pv, sh3),
                           mask=mm)
        mb = local * METADATA_LEN
        chip = lax.shift_right_logical(pv, jnp.int32(10))
        tok = lax.shift_right_logical(pv, sh3) & (SEQ - 1)
        topk = pv & (TOP_K - 1)
        plsc.store_scatter(meta_v, [mb], chip, mask=mm)
        plsc.store_scatter(meta_v, [mb + 1], tok, mask=mm)
        plsc.store_scatter(meta_v, [mb + 2], topk, mask=mm)
        plsc.store_scatter(meta_v, [mb + 3], ev, mask=mm)
        plsc.store_scatter(meta_v, [mb + 4], wv, mask=mm)
        plsc.store_scatter(meta_v, [mb + 5], zero, mask=mm)
        plsc.store_scatter(meta_v, [mb + 6], zero, mask=mm)
        plsc.store_scatter(meta_v, [mb + 7], zero, mask=mm)
        r0 = r0 + plsc.all_reduce_population_count(m0r)
        r1 = r1 + plsc.all_reduce_population_count(m1r)
        return r0, r1

    nq = lax.shift_right_logical(n + jnp.int32(LANES - 1), jnp.int32(4))
    r0, r1 = lax.fori_loop(jnp.int32(0), nq, scan_b, (zero, zero))

    cnt_v[...] = jnp.where(iota == 0, r0, jnp.where(iota == 1, r1, 0))
    pltpu.sync_copy(inv_v, inv_hbm.at[pl.ds(w * SLOTS_PW, SLOTS_PW)])
    pltpu.sync_copy(
        meta_v,
        meta_hbm.at[pl.ds(w * SLOTS_PW * METADATA_LEN,
                          SLOTS_PW * METADATA_LEN)])
    pltpu.sync_copy(cnt_v, cnt_hbm.at[w])


_EXP_PER_STEP = 4


def _gather_body(inv_ref, x_ref, out_ref):
    b = pl.program_id(0)

    base = b * _EXP_PER_STEP * MAX_DISP
    for a in range(_EXP_PER_STEP * MAX_DISP // 8):
        rows = [x_ref[inv_ref[base + 8 * a + k]] for k in range(8)]
        blk = jnp.stack(rows, axis=0)
        out_ref[0, (8 * a) // MAX_DISP, pl.ds((8 * a) % MAX_DISP, 8), :] = (
            blk.reshape(8, HIDDEN))


def _gather_tc(inv, x_aug):
    steps_per_chip = EXPERTS_PER_CHIP // _EXP_PER_STEP

    def _out_map(i, inv_s):
        i = lax.convert_element_type(i, jnp.int32)
        return (i // jnp.int32(steps_per_chip),
                i % jnp.int32(steps_per_chip), jnp.int32(0), jnp.int32(0))

    return pl.pallas_call(
        _gather_body,
        grid_spec=pltpu.PrefetchScalarGridSpec(
            num_scalar_prefetch=1,
            grid=(N_EXPERTS // _EXP_PER_STEP,),
            in_specs=[
                pl.BlockSpec(
                    (X_ROWS + 8, HIDDEN // 128, 128),
                    lambda i, inv_s: (jnp.int32(0), jnp.int32(0),
                                      jnp.int32(0))),
            ],
            out_specs=pl.BlockSpec((1, _EXP_PER_STEP, MAX_DISP, HIDDEN),
                                   _out_map),
        ),
        out_shape=jax.ShapeDtypeStruct(
            (NUM_CHIPS, EXPERTS_PER_CHIP, MAX_DISP, HIDDEN), jnp.float32),
    )(inv, x_aug)


def kernel(x, weights, indices):
    idx32 = indices.astype(jnp.int32).reshape(-1)
    wb = lax.bitcast_convert_type(
        weights.astype(jnp.bfloat16), jnp.int16).astype(jnp.int32).reshape(-1)
    inv, metaf, cnt = _route_sc(idx32, wb)
    x_aug = jnp.concatenate(
        [x.reshape(X_ROWS, HIDDEN // 128, 128),
         jnp.zeros((8, HIDDEN // 128, 128), jnp.float32)])
    buf = _gather_tc(inv, x_aug)
    meta = metaf.reshape(NUM_CHIPS, EXPERTS_PER_CHIP, MAX_DISP, METADATA_LEN)
    counter = cnt[:, :EPW].reshape(NUM_CHIPS, EXPERTS_PER_CHIP)
    return buf, meta, counter


# speedup vs baseline: 1.0101x; 1.0101x over previous
import functools

import jax
import jax.numpy as jnp
from jax import lax
from jax.experimental import pallas as pl
from jax.experimental.pallas import tpu as pltpu
from jax.experimental.pallas import tpu_sc as plsc

NUM_CHIPS = 8
EXPERTS_PER_CHIP = 8
N_EXPERTS = 64
TOP_K = 8
METADATA_LEN = 8
MAX_DISP = 256
SEQ = 128
HIDDEN = 2048

N_PICKS = NUM_CHIPS * SEQ * TOP_K
N_SLOTS = N_EXPERTS * MAX_DISP
LANES = 16
NW = 32
EPW = N_EXPERTS // NW
SLOTS_PW = EPW * MAX_DISP
X_ROWS = NUM_CHIPS * SEQ
SENTINEL = X_ROWS

_sc_mesh = plsc.VectorSubcoreMesh(core_axis_name="c", subcore_axis_name="s")


@functools.partial(
    pl.kernel,
    out_type=(
        jax.ShapeDtypeStruct((N_SLOTS,), jnp.int32),
        jax.ShapeDtypeStruct((N_SLOTS * METADATA_LEN,), jnp.int32),
        jax.ShapeDtypeStruct((NW * LANES,), jnp.int32),
    ),
    mesh=_sc_mesh,
    compiler_params=pltpu.CompilerParams(needs_layout_passes=False),
    scratch_types=(
        pltpu.VMEM((N_PICKS,), jnp.int32),
        pltpu.VMEM((N_PICKS,), jnp.int32),
        pltpu.VMEM((SLOTS_PW,), jnp.int32),
        pltpu.VMEM((SLOTS_PW * METADATA_LEN,), jnp.int32),
        pltpu.VMEM((LANES,), jnp.int32),
        pltpu.VMEM((N_PICKS,), jnp.int32),
    ),
)
def _route_sc(idx_hbm, wb_hbm, inv_hbm, meta_hbm, cnt_hbm,
              idx_v, wb_v, inv_v, meta_v, cnt_v, pk_v):
    w = lax.axis_index("s") * 2 + lax.axis_index("c")
    pltpu.sync_copy(idx_hbm, idx_v)
    pltpu.sync_copy(wb_hbm, wb_v)

    sent = jnp.full((LANES,), SENTINEL, jnp.int32)
    neg1 = jnp.full((LANES,), -1, jnp.int32)
    zero = jnp.zeros((LANES,), jnp.int32)
    iota = lax.iota(jnp.int32, LANES)

    def init_inv(i, c):
        inv_v[pl.ds(i * jnp.int32(LANES), LANES)] = sent
        return c
    lax.fori_loop(jnp.int32(0), jnp.int32(SLOTS_PW // LANES), init_inv, 0)

    def init_meta(i, c):
        meta_v[pl.ds(i * jnp.int32(LANES), LANES)] = neg1
        return c
    lax.fori_loop(jnp.int32(0), jnp.int32(SLOTS_PW * METADATA_LEN // LANES),
                  init_meta, 0)

    e0 = lax.convert_element_type(w, jnp.int32) * jnp.int32(EPW)
    wid = lax.convert_element_type(w, jnp.int32)

    def scan_a(p, off):
        base0 = p * jnp.int32(4 * LANES)
        for u in range(4):
            base = base0 + jnp.int32(u * LANES)
            ev = idx_v[pl.ds(base, LANES)]
            mm = lax.shift_right_logical(ev, jnp.int32(1)) == wid
            mi = jnp.where(mm, jnp.int32(1), jnp.int32(0))
            pos = off + plsc.cumsum(mi) - mi
            plsc.store_scatter(pk_v, [pos], base + iota, mask=mm)
            off = off + plsc.all_reduce_population_count(mm)
        return off

    offv = lax.fori_loop(jnp.int32(0), jnp.int32(N_PICKS // LANES // 4),
                         scan_a, zero)
    n = lax.reduce_max(offv, axes=(0,))

    def scan_b(q, carry):
        r0, r1 = carry
        base = q * jnp.int32(LANES)
        vl = (base + iota) < n
        pv = pk_v[pl.ds(base, LANES)]
        ev = plsc.load_gather(idx_v, [pv], mask=vl)
        wv = plsc.load_gather(wb_v, [pv], mask=vl)
        m0r = vl & (ev == e0)
        m1r = vl & (ev == e0 + 1)
        i0 = jnp.where(m0r, jnp.int32(1), jnp.int32(0))
        i1 = jnp.where(m1r, jnp.int32(1), jnp.int32(0))
        s0 = r0 + plsc.cumsum(i0) - i0
        s1 = r1 + plsc.cumsum(i1) - i1
        m0 = m0r & (s0 < MAX_DISP)
        m1 = m1r & (s1 < MAX_DISP)
        mm = m0 | m1
        local = jnp.where(m0, s0, s1 + MAX_DISP)
        sh3 = jnp.int32(3)
        plsc.store_scatter(inv_v, [local], lax.shift_right_logical(pv, sh3),
                           mask=mm)
        mb = local * METADATA_LEN
        chip = lax.shift_right_logical(pv, jnp.int32(10))
        tok = lax.shift_right_logical(pv, sh3) & (SEQ - 1)
        topk = pv & (TOP_K - 1)
        plsc.store_scatter(meta_v, [mb], chip, mask=mm)
        plsc.store_scatter(meta_v, [mb + 1], tok, mask=mm)
        plsc.store_scatter(meta_v, [mb + 2], topk, mask=mm)
        plsc.store_scatter(meta_v, [mb + 3], ev, mask=mm)
        plsc.store_scatter(meta_v, [mb + 4], wv, mask=mm)
        plsc.store_scatter(meta_v, [mb + 5], zero, mask=mm)
        plsc.store_scatter(meta_v, [mb + 6], zero, mask=mm)
        plsc.store_scatter(meta_v, [mb + 7], zero, mask=mm)
        r0 = r0 + plsc.all_reduce_population_count(m0r)
        r1 = r1 + plsc.all_reduce_population_count(m1r)
        return r0, r1

    nq = lax.shift_right_logical(n + jnp.int32(LANES - 1), jnp.int32(4))
    r0, r1 = lax.fori_loop(jnp.int32(0), nq, scan_b, (zero, zero))

    cnt_v[...] = jnp.where(iota == 0, r0, jnp.where(iota == 1, r1, 0))
    pltpu.sync_copy(inv_v, inv_hbm.at[pl.ds(w * SLOTS_PW, SLOTS_PW)])
    pltpu.sync_copy(
        meta_v,
        meta_hbm.at[pl.ds(w * SLOTS_PW * METADATA_LEN,
                          SLOTS_PW * METADATA_LEN)])
    pltpu.sync_copy(cnt_v, cnt_hbm.at[pl.ds(w * LANES, LANES)])


_EXP_PER_STEP = 4


def _gather_body(inv_ref, x_ref, out_ref):
    b = pl.program_id(0)

    base = b * _EXP_PER_STEP * MAX_DISP
    for a in range(_EXP_PER_STEP * MAX_DISP // 8):
        rows = [x_ref[inv_ref[base + 8 * a + k]] for k in range(8)]
        blk = jnp.stack(rows, axis=0)
        out_ref[0, (8 * a) // MAX_DISP, pl.ds((8 * a) % MAX_DISP, 8), :] = (
            blk.reshape(8, HIDDEN))


def _gather_tc(inv, x_aug):
    steps_per_chip = EXPERTS_PER_CHIP // _EXP_PER_STEP

    def _out_map(i, inv_s):
        i = lax.convert_element_type(i, jnp.int32)
        return (i // jnp.int32(steps_per_chip),
                i % jnp.int32(steps_per_chip), jnp.int32(0), jnp.int32(0))

    return pl.pallas_call(
        _gather_body,
        grid_spec=pltpu.PrefetchScalarGridSpec(
            num_scalar_prefetch=1,
            grid=(N_EXPERTS // _EXP_PER_STEP,),
            in_specs=[
                pl.BlockSpec(
                    (X_ROWS + 8, HIDDEN // 128, 128),
                    lambda i, inv_s: (jnp.int32(0), jnp.int32(0),
                                      jnp.int32(0))),
            ],
            out_specs=pl.BlockSpec((1, _EXP_PER_STEP, MAX_DISP, HIDDEN),
                                   _out_map),
        ),
        out_shape=jax.ShapeDtypeStruct(
            (NUM_CHIPS, EXPERTS_PER_CHIP, MAX_DISP, HIDDEN), jnp.float32),
    )(inv, x_aug)


def kernel(x, weights, indices):
    idx32 = indices.astype(jnp.int32).reshape(-1)
    wb = lax.bitcast_convert_type(
        weights.astype(jnp.bfloat16), jnp.int16).astype(jnp.int32).reshape(-1)
    inv, metaf, cnt = _route_sc(idx32, wb)
    x_aug = jnp.concatenate(
        [x.reshape(X_ROWS, HIDDEN // 128, 128),
         jnp.zeros((8, HIDDEN // 128, 128), jnp.float32)])
    buf = _gather_tc(inv, x_aug)
    meta = metaf.reshape(NUM_CHIPS, EXPERTS_PER_CHIP, MAX_DISP, METADATA_LEN)
    counter = cnt.reshape(NW, LANES)[:, :EPW].reshape(
        NUM_CHIPS, EXPERTS_PER_CHIP)
    return buf, meta, counter
